# Initial kernel scaffold; baseline (speedup 1.0000x reference)
import functools

import jax
import jax.numpy as jnp
from jax import lax
from jax.experimental import pallas as pl
from jax.experimental.pallas import tpu as pltpu
from jax.experimental.pallas import tpu_sc as plsc

S = 2048
HID = 1024
INTER = 2048
E = 8
K = 2
N = S * K
BLK = 256
G = N // BLK + E
NSLOTS = G * BLK

NWORKERS = 32
GATHER_CHUNK = 96
TOK_PER_WORKER = S // NWORKERS
TOK_CHUNK = 16


def _routing_metadata(expert_indices, expert_weights):
    idx_flat = expert_indices.reshape(N).astype(jnp.int32)
    w_flat = expert_weights.reshape(N)
    order = jnp.argsort(idx_flat, stable=True).astype(jnp.int32)
    sorted_e = idx_flat[order]
    counts = jnp.zeros((E,), jnp.int32).at[idx_flat].add(1)
    starts = jnp.concatenate([jnp.zeros((1,), jnp.int32),
                              jnp.cumsum(counts)[:-1].astype(jnp.int32)])
    nblk = (counts + BLK - 1) // BLK
    blk_starts = jnp.concatenate([jnp.zeros((1,), jnp.int32),
                                  jnp.cumsum(nblk)[:-1].astype(jnp.int32)])
    pad_start = blk_starts * BLK
    p = jnp.arange(N, dtype=jnp.int32)
    slot_of_sorted = pad_start[sorted_e] + p - starts[sorted_e]
    gather_tok = jnp.zeros((NSLOTS,), jnp.int32).at[slot_of_sorted].set(order // K)
    slot_w = jnp.zeros((NSLOTS,), jnp.float32).at[slot_of_sorted].set(w_flat[order])
    block_expert = jnp.clip(
        jnp.searchsorted(blk_starts, jnp.arange(G, dtype=jnp.int32), side="right")
        .astype(jnp.int32) - 1, 0, E - 1)
    pair_slot = jnp.zeros((N,), jnp.int32).at[order].set(slot_of_sorted)
    return gather_tok, slot_w, block_expert, pair_slot


def _sc_mesh():
    return plsc.VectorSubcoreMesh(core_axis_name="c", subcore_axis_name="s")


@functools.partial(
    pl.kernel,
    out_type=jax.ShapeDtypeStruct((NSLOTS, HID), jnp.float32),
    mesh=_sc_mesh(),
    scratch_types=[
        pltpu.VMEM((GATHER_CHUNK,), jnp.int32),
        pltpu.VMEM((GATHER_CHUNK, HID), jnp.float32),
        pltpu.SemaphoreType.DMA,
    ],
)
def _sc_gather(x_hbm, ids_hbm, out_hbm, idx_v, rows_v, sem):
    wid = lax.axis_index("s") * 2 + lax.axis_index("c")
    base = wid * (NSLOTS // NWORKERS)
    for c in range(NSLOTS // NWORKERS // GATHER_CHUNK):
        off = base + c * GATHER_CHUNK
        pltpu.sync_copy(ids_hbm.at[pl.ds(off, GATHER_CHUNK)], idx_v)
        pltpu.async_copy(x_hbm.at[idx_v], rows_v, sem).wait()
        pltpu.sync_copy(rows_v, out_hbm.at[pl.ds(off, GATHER_CHUNK)])


@functools.partial(
    pl.kernel,
    out_type=jax.ShapeDtypeStruct((S, HID), jnp.float32),
    mesh=_sc_mesh(),
    scratch_types=[
        pltpu.VMEM((K * TOK_CHUNK,), jnp.int32),
        pltpu.VMEM((K * TOK_CHUNK, HID), jnp.float32),
        pltpu.VMEM((TOK_CHUNK, HID), jnp.float32),
        pltpu.SemaphoreType.DMA,
    ],
)
def _sc_combine(y_hbm, pair_hbm, out_hbm, idx_v, rows_v, acc_v, sem):
    wid = lax.axis_index("s") * 2 + lax.axis_index("c")
    tbase = wid * TOK_PER_WORKER
    for c in range(TOK_PER_WORKER // TOK_CHUNK):
        t0 = tbase + c * TOK_CHUNK
        pltpu.sync_copy(pair_hbm.at[pl.ds(t0 * K, K * TOK_CHUNK)], idx_v)
        pltpu.async_copy(y_hbm.at[idx_v], rows_v, sem).wait()

        def col_body(j, carry):
            cs = pl.ds(j * 16, 16)
            for i in range(TOK_CHUNK):
                acc_v[i, cs] = rows_v[2 * i, cs] + rows_v[2 * i + 1, cs]
            return carry

        lax.fori_loop(0, HID // 16, col_body, 0)
        pltpu.sync_copy(acc_v, out_hbm.at[pl.ds(t0, TOK_CHUNK)])


def _ffn_body(be_ref, xs_ref, g_ref, u_ref, d_ref, w_ref, o_ref):
    xb = xs_ref[...]
    gate = lax.dot_general(xb, g_ref[0], (((1,), (1,)), ((), ())),
                           preferred_element_type=jnp.float32)
    up = lax.dot_general(xb, u_ref[0], (((1,), (1,)), ((), ())),
                         preferred_element_type=jnp.float32)
    h = (gate * jax.nn.sigmoid(gate)) * up
    y = lax.dot_general(h, d_ref[0], (((1,), (1,)), ((), ())),
                        preferred_element_type=jnp.float32)
    o_ref[...] = y * w_ref[0, 0][:, None]


def _tc_ffn(x_sorted, gate_proj, up_proj, down_proj, slot_w, block_expert):
    grid_spec = pltpu.PrefetchScalarGridSpec(
        num_scalar_prefetch=1,
        grid=(G,),
        in_specs=[
            pl.BlockSpec((BLK, HID), lambda g, be: (g, 0)),
            pl.BlockSpec((1, INTER, HID), lambda g, be: (be[g], 0, 0)),
            pl.BlockSpec((1, INTER, HID), lambda g, be: (be[g], 0, 0)),
            pl.BlockSpec((1, HID, INTER), lambda g, be: (be[g], 0, 0)),
            pl.BlockSpec((1, 1, BLK), lambda g, be: (g, 0, 0)),
        ],
        out_specs=pl.BlockSpec((BLK, HID), lambda g, be: (g, 0)),
    )
    return pl.pallas_call(
        _ffn_body,
        grid_spec=grid_spec,
        out_shape=jax.ShapeDtypeStruct((NSLOTS, HID), jnp.float32),
        compiler_params=pltpu.CompilerParams(
            vmem_limit_bytes=100 * 1024 * 1024,
        ),
    )(block_expert, x_sorted, gate_proj, up_proj, down_proj,
      slot_w.reshape(G, 1, BLK))


def kernel(x, expert_indices, expert_weights, gate_proj, up_proj, down_proj):
    batch, seq, hid = x.shape
    x2d = x.reshape(S, HID)
    gather_tok, slot_w, block_expert, pair_slot = _routing_metadata(
        expert_indices, expert_weights)
    x_sorted = _sc_gather(x2d, gather_tok)
    y_sorted = _tc_ffn(x_sorted, gate_proj, up_proj, down_proj,
                       slot_w, block_expert)
    out = _sc_combine(y_sorted, pair_slot)
    return out.reshape(batch, seq, hid)


# speedup vs baseline: 1.1111x; 1.1111x over previous
import functools

import jax
import jax.numpy as jnp
from jax import lax
from jax.experimental import pallas as pl
from jax.experimental.pallas import tpu as pltpu
from jax.experimental.pallas import tpu_sc as plsc

S = 2048
HID = 1024
INTER = 2048
E = 8
K = 2
N = S * K
BLK = 256
G = N // BLK + E
NSLOTS = G * BLK

NWORKERS = 32
GATHER_CHUNK = 96
TOK_PER_WORKER = S // NWORKERS
TOK_CHUNK = 16


def _routing_metadata(expert_indices, expert_weights):
    idx_flat = expert_indices.reshape(N).astype(jnp.int32)
    w_flat = expert_weights.reshape(N)
    order = jnp.argsort(idx_flat, stable=True).astype(jnp.int32)
    sorted_e = idx_flat[order]
    counts = jnp.zeros((E,), jnp.int32).at[idx_flat].add(1)
    starts = jnp.concatenate([jnp.zeros((1,), jnp.int32),
                              jnp.cumsum(counts)[:-1].astype(jnp.int32)])
    nblk = (counts + BLK - 1) // BLK
    blk_starts = jnp.concatenate([jnp.zeros((1,), jnp.int32),
                                  jnp.cumsum(nblk)[:-1].astype(jnp.int32)])
    pad_start = blk_starts * BLK
    p = jnp.arange(N, dtype=jnp.int32)
    slot_of_sorted = pad_start[sorted_e] + p - starts[sorted_e]
    gather_tok = jnp.zeros((NSLOTS,), jnp.int32).at[slot_of_sorted].set(order // K)
    slot_w = jnp.zeros((NSLOTS,), jnp.float32).at[slot_of_sorted].set(w_flat[order])
    block_expert = jnp.clip(
        jnp.searchsorted(blk_starts, jnp.arange(G, dtype=jnp.int32), side="right")
        .astype(jnp.int32) - 1, 0, E - 1)
    pair_slot = jnp.zeros((N,), jnp.int32).at[order].set(slot_of_sorted)
    return gather_tok, slot_w, block_expert, pair_slot


def _sc_gather_body(x_hbm, ids_hbm, out_hbm, idx_v, rows_v, sem):
    wid = lax.axis_index("s") * 2 + lax.axis_index("c")
    base = wid * (NSLOTS // NWORKERS)
    for c in range(NSLOTS // NWORKERS // GATHER_CHUNK):
        off = base + c * GATHER_CHUNK
        pltpu.sync_copy(ids_hbm.at[pl.ds(off, GATHER_CHUNK)], idx_v)
        pltpu.async_copy(x_hbm.at[idx_v], rows_v, sem).wait()
        pltpu.sync_copy(rows_v, out_hbm.at[pl.ds(off, GATHER_CHUNK)])


def _sc_combine_body(y_hbm, pair_hbm, out_hbm, idx_v, rows_v, acc_v, sem):
    wid = lax.axis_index("s") * 2 + lax.axis_index("c")
    tbase = wid * TOK_PER_WORKER
    for c in range(TOK_PER_WORKER // TOK_CHUNK):
        t0 = tbase + c * TOK_CHUNK
        pltpu.sync_copy(pair_hbm.at[pl.ds(t0 * K, K * TOK_CHUNK)], idx_v)
        pltpu.async_copy(y_hbm.at[idx_v], rows_v, sem).wait()

        def col_body(j, carry):
            cs = pl.ds(j * 16, 16)
            for i in range(TOK_CHUNK):
                acc_v[i, cs] = rows_v[2 * i, cs] + rows_v[2 * i + 1, cs]
            return carry

        lax.fori_loop(0, HID // 16, col_body, 0)
        pltpu.sync_copy(acc_v, out_hbm.at[pl.ds(t0, TOK_CHUNK)])


@functools.lru_cache(maxsize=None)
def _build_sc_kernels():
    mesh = plsc.VectorSubcoreMesh(core_axis_name="c", subcore_axis_name="s")
    gather = pl.kernel(
        _sc_gather_body,
        out_type=jax.ShapeDtypeStruct((NSLOTS, HID), jnp.float32),
        mesh=mesh,
        scratch_types=[
            pltpu.VMEM((GATHER_CHUNK,), jnp.int32),
            pltpu.VMEM((GATHER_CHUNK, HID), jnp.float32),
            pltpu.SemaphoreType.DMA,
        ],
    )
    combine = pl.kernel(
        _sc_combine_body,
        out_type=jax.ShapeDtypeStruct((S, HID), jnp.float32),
        mesh=mesh,
        scratch_types=[
            pltpu.VMEM((K * TOK_CHUNK,), jnp.int32),
            pltpu.VMEM((K * TOK_CHUNK, HID), jnp.float32),
            pltpu.VMEM((TOK_CHUNK, HID), jnp.float32),
            pltpu.SemaphoreType.DMA,
        ],
    )
    return gather, combine


def _ffn_body(be_ref, xs_ref, g_ref, u_ref, d_ref, w_ref, o_ref):
    xb = xs_ref[...]
    gate = lax.dot_general(xb, g_ref[0], (((1,), (1,)), ((), ())),
                           preferred_element_type=jnp.float32)
    up = lax.dot_general(xb, u_ref[0], (((1,), (1,)), ((), ())),
                         preferred_element_type=jnp.float32)
    h = (gate * jax.nn.sigmoid(gate)) * up
    y = lax.dot_general(h, d_ref[0], (((1,), (1,)), ((), ())),
                        preferred_element_type=jnp.float32)
    o_ref[...] = y * w_ref[0, 0][:, None]


def _tc_ffn(x_sorted, gate_proj, up_proj, down_proj, slot_w, block_expert):
    grid_spec = pltpu.PrefetchScalarGridSpec(
        num_scalar_prefetch=1,
        grid=(G,),
        in_specs=[
            pl.BlockSpec((BLK, HID), lambda g, be: (g, 0)),
            pl.BlockSpec((1, INTER, HID), lambda g, be: (be[g], 0, 0)),
            pl.BlockSpec((1, INTER, HID), lambda g, be: (be[g], 0, 0)),
            pl.BlockSpec((1, HID, INTER), lambda g, be: (be[g], 0, 0)),
            pl.BlockSpec((1, 1, BLK), lambda g, be: (g, 0, 0)),
        ],
        out_specs=pl.BlockSpec((BLK, HID), lambda g, be: (g, 0)),
    )
    return pl.pallas_call(
        _ffn_body,
        grid_spec=grid_spec,
        out_shape=jax.ShapeDtypeStruct((NSLOTS, HID), jnp.float32),
        compiler_params=pltpu.CompilerParams(
            vmem_limit_bytes=100 * 1024 * 1024,
        ),
    )(block_expert, x_sorted, gate_proj, up_proj, down_proj,
      slot_w.reshape(G, 1, BLK))


def kernel(x, expert_indices, expert_weights, gate_proj, up_proj, down_proj):
    batch, seq, hid = x.shape
    x2d = x.reshape(S, HID)
    gather_tok, slot_w, block_expert, pair_slot = _routing_metadata(
        expert_indices, expert_weights)
    sc_gather, sc_combine = _build_sc_kernels()
    x_sorted = sc_gather(x2d, gather_tok)
    y_sorted = _tc_ffn(x_sorted, gate_proj, up_proj, down_proj,
                       slot_w, block_expert)
    out = sc_combine(y_sorted, pair_slot)
    return out.reshape(batch, seq, hid)
